# Initial kernel scaffold; baseline (speedup 1.0000x reference)
import jax
import jax.numpy as jnp
from jax.experimental import pallas as pl


def kernel(x_categ, table, offsets):
    raise NotImplementedError("write your pallas kernel here")


# speedup vs baseline: 1.4756x; 1.4756x over previous
import functools

import jax
import jax.numpy as jnp
from jax import lax
from jax.experimental import pallas as pl
from jax.experimental.pallas import tpu as pltpu
from jax.experimental.pallas import tpu_sc as plsc

NC = 2
NS = 16
NW = NC * NS

B = 16384
F = 26
DIM = 32
ROWS = B * F
RPW = ROWS // NW
CHUNK = 128
NCHUNK = RPW // CHUNK


def _body(x_hbm, off_hbm, table_hbm, out_hbm, idx_v, off_v, rows_v, gsem, ssem):
    wid = lax.axis_index("s") * NC + lax.axis_index("c")

    pltpu.sync_copy(x_hbm.at[wid], idx_v)
    pltpu.sync_copy(off_hbm, off_v)

    def add_row(j, carry):
        for t in range(CHUNK // 16):
            sl = pl.ds(t * 16, 16)
            idx_v[j, sl] = idx_v[j, sl] + off_v[j, sl]
        return carry

    lax.fori_loop(0, NCHUNK, add_row, 0)

    def do_chunk(j, carry):
        pltpu.async_copy(table_hbm.at[idx_v.at[j]], rows_v, gsem).wait()
        pltpu.async_copy(rows_v, out_hbm.at[wid * NCHUNK + j], ssem).wait()
        return carry

    lax.fori_loop(0, NCHUNK, do_chunk, 0)


@functools.partial(jax.jit, static_argnames=())
def _run(x_resh, off_tiled, table):
    mesh = plsc.VectorSubcoreMesh(
        core_axis_name="c", subcore_axis_name="s", num_cores=NC, num_subcores=NS
    )
    fn = pl.kernel(
        _body,
        out_type=jax.ShapeDtypeStruct((NW * NCHUNK, CHUNK, DIM), jnp.float32),
        mesh=mesh,
        scratch_types=[
            pltpu.VMEM((NCHUNK, CHUNK), jnp.int32),
            pltpu.VMEM((NCHUNK, CHUNK), jnp.int32),
            pltpu.VMEM((CHUNK, DIM), jnp.float32),
            pltpu.SemaphoreType.DMA,
            pltpu.SemaphoreType.DMA,
        ],
        compiler_params=pltpu.CompilerParams(use_tc_tiling_on_sc=False),
    )
    return fn(x_resh, off_tiled, table)


def kernel(x_categ, table, offsets):
    x = x_categ.astype(jnp.int32).reshape(NW, NCHUNK, CHUNK)
    off_tiled = jnp.tile(offsets.astype(jnp.int32), RPW // F).reshape(NCHUNK, CHUNK)
    out = _run(x, off_tiled, table)
    return out.reshape(B, F, DIM)


# speedup vs baseline: 1.6268x; 1.1025x over previous
import functools

import jax
import jax.numpy as jnp
from jax import lax
from jax.experimental import pallas as pl
from jax.experimental.pallas import tpu as pltpu
from jax.experimental.pallas import tpu_sc as plsc

NC = 2
NS = 16
NW = NC * NS

B = 16384
F = 26
DIM = 32
ROWS = B * F
RPW = ROWS // NW
CHUNK = 128
NCHUNK = RPW // CHUNK


NBUF = 16
LOOKAHEAD = 8
STORE_LAG = NBUF - LOOKAHEAD - 1


def _add_offsets(idx_v, off_v, j):
    for t in range(CHUNK // 16):
        sl = pl.ds(t * 16, 16)
        idx_v[j, sl] = idx_v[j, sl] + off_v[j, sl]


def _body(x_hbm, off_hbm, table_hbm, out_hbm, idx_v, off_v, rows_v, gsem, ssem):
    wid = lax.axis_index("s") * NC + lax.axis_index("c")
    cbase = wid * NCHUNK

    pltpu.sync_copy(x_hbm.at[wid], idx_v)
    pltpu.sync_copy(off_hbm, off_v)

    for b in range(LOOKAHEAD):
        _add_offsets(idx_v, off_v, b)
        pltpu.async_copy(table_hbm.at[idx_v.at[b]], rows_v.at[b], gsem)

    def step(j, carry):
        b = j & (NBUF - 1)
        pltpu.make_async_copy(table_hbm.at[idx_v.at[j]], rows_v.at[b], gsem).wait()
        pltpu.async_copy(rows_v.at[b], out_hbm.at[cbase + j], ssem)

        @pl.when(j >= STORE_LAG)
        def _drain_one_store():
            pltpu.make_async_copy(rows_v.at[0], out_hbm.at[cbase], ssem).wait()

        @pl.when(j + LOOKAHEAD < NCHUNK)
        def _launch_next_gather():
            nj = j + LOOKAHEAD
            _add_offsets(idx_v, off_v, nj)
            pltpu.async_copy(
                table_hbm.at[idx_v.at[nj]], rows_v.at[nj & (NBUF - 1)], gsem
            )

        return carry

    lax.fori_loop(0, NCHUNK, step, 0)

    for _ in range(STORE_LAG):
        pltpu.make_async_copy(rows_v.at[0], out_hbm.at[cbase], ssem).wait()


@functools.partial(jax.jit, static_argnames=())
def _run(x_resh, off_tiled, table):
    mesh = plsc.VectorSubcoreMesh(
        core_axis_name="c", subcore_axis_name="s", num_cores=NC, num_subcores=NS
    )
    fn = pl.kernel(
        _body,
        out_type=jax.ShapeDtypeStruct((NW * NCHUNK, CHUNK, DIM), jnp.float32),
        mesh=mesh,
        scratch_types=[
            pltpu.VMEM((NCHUNK, CHUNK), jnp.int32),
            pltpu.VMEM((NCHUNK, CHUNK), jnp.int32),
            pltpu.VMEM((NBUF, CHUNK, DIM), jnp.float32),
            pltpu.SemaphoreType.DMA,
            pltpu.SemaphoreType.DMA,
        ],
        compiler_params=pltpu.CompilerParams(use_tc_tiling_on_sc=False),
    )
    return fn(x_resh, off_tiled, table)


def kernel(x_categ, table, offsets):
    x = x_categ.astype(jnp.int32).reshape(NW, NCHUNK, CHUNK)
    off_tiled = jnp.tile(offsets.astype(jnp.int32), RPW // F).reshape(NCHUNK, CHUNK)
    out = _run(x, off_tiled, table)
    return out.reshape(B, F, DIM)
